# Initial kernel scaffold; baseline (speedup 1.0000x reference)
import jax
import jax.numpy as jnp
from jax.experimental import pallas as pl


def kernel(X, edge_index, edge_weight, edge_attr, W_proj, b_proj, Ws1, b1, Ws2, b2, Ws3, b3, Wt0, Wt1, Wt2, Wt3, Wt4, Wt5):
    raise NotImplementedError("write your pallas kernel here")


# speedup vs baseline: 43.0124x; 43.0124x over previous
import functools

import jax
import jax.numpy as jnp
import numpy as np
from jax import lax
from jax.experimental import pallas as pl
from jax.experimental.pallas import tpu as pltpu
from jax.experimental.pallas import tpu_sc as plsc

F32 = jnp.float32
_CUT = 5.0


def _a_body(xt_ref, wp_ref, bp_ref, w0_ref, w1_ref, w2_ref,
            f1_ref, f2_ref, xn_ref):
    wp = wp_ref[...]
    bp = bp_ref[...]
    xp = [jnp.dot(xt_ref[p], wp, preferred_element_type=F32) + bp
          for p in range(9)]
    nrm = xp[0] * xp[0]
    for p in range(1, 9):
        nrm = nrm + xp[p] * xp[p]
    inv = 1.0 / (jnp.maximum(nrm, 0.01) + 1.0)
    xn = [xp[p] * inv for p in range(9)]
    for p in range(9):
        xn_ref[p] = xn[p]
    third = jnp.float32(1.0 / 3.0)
    iso = (xn[0] + xn[4] + xn[8]) * third
    a01 = 0.5 * (xn[1] - xn[3])
    a02 = 0.5 * (xn[2] - xn[6])
    a12 = 0.5 * (xn[5] - xn[7])
    s00 = xn[0] - iso
    s01 = 0.5 * (xn[1] + xn[3])
    s02 = 0.5 * (xn[2] + xn[6])
    s11 = xn[4] - iso
    s12 = 0.5 * (xn[5] + xn[7])
    w0 = w0_ref[...]
    w1 = w1_ref[...]
    w2 = w2_ref[...]
    mm = lambda x, w: jnp.dot(x, w, preferred_element_type=F32)
    f1_ref[...] = jnp.concatenate(
        [mm(iso, w0), mm(a01, w1), mm(a02, w1), mm(a12, w1)], axis=1)
    f2_ref[...] = jnp.concatenate(
        [mm(s00, w2), mm(s01, w2), mm(s02, w2), mm(s11, w2), mm(s12, w2)],
        axis=1)


def _run_a(Xt, W_proj, b_proj, Wt0, Wt1, Wt2, Bn):
    n = Xt.shape[1]
    grid = n // Bn
    full = lambda *s: pl.BlockSpec(s, lambda i: tuple(0 for _ in s))
    return pl.pallas_call(
        _a_body,
        grid=(grid,),
        in_specs=[
            pl.BlockSpec((9, Bn, 128), lambda i: (0, i, 0)),
            full(128, 32), full(1, 32), full(32, 32), full(32, 32),
            full(32, 32),
        ],
        out_specs=[
            pl.BlockSpec((Bn, 128), lambda i: (i, 0)),
            pl.BlockSpec((Bn, 160), lambda i: (i, 0)),
            pl.BlockSpec((9, Bn, 32), lambda i: (0, i, 0)),
        ],
        out_shape=[
            jax.ShapeDtypeStruct((n, 128), F32),
            jax.ShapeDtypeStruct((n, 160), F32),
            jax.ShapeDtypeStruct((9, n, 32), F32),
        ],
    )(Xt, W_proj, b_proj, Wt0, Wt1, Wt2)


def _b_body(attr_ref, ew_ref, w1_ref, b1_ref, w2_ref, b2_ref, w3_ref, b3_ref,
            ea1_ref, ea2_ref):
    mm = lambda x, w: jnp.dot(x, w, preferred_element_type=F32)
    silu = lambda x: x * jax.nn.sigmoid(x)
    h = silu(mm(attr_ref[...], w1_ref[...]) + b1_ref[...])
    h = silu(mm(h, w2_ref[...]) + b2_ref[...])
    h = silu(mm(h, w3_ref[...]) + b3_ref[...])
    d = ew_ref[...]
    c = jnp.where(d < _CUT, 0.5 * (jnp.cos(d * (np.pi / _CUT)) + 1.0), 0.0)
    h = h * c
    ea1_ref[...] = h[:, :64]
    ea2_ref[...] = h[:, 64:96]


def _run_b(edge_attr, ew, Ws1, b1, Ws2, b2, Ws3p, b3p, Be):
    e = edge_attr.shape[0]
    grid = e // Be
    full = lambda *s: pl.BlockSpec(s, lambda i: tuple(0 for _ in s))
    return pl.pallas_call(
        _b_body,
        grid=(grid,),
        in_specs=[
            pl.BlockSpec((Be, 32), lambda i: (i, 0)),
            pl.BlockSpec((Be, 1), lambda i: (i, 0)),
            full(32, 32), full(1, 32), full(32, 64), full(1, 64),
            full(64, 96), full(1, 96),
        ],
        out_specs=[
            pl.BlockSpec((Be, 64), lambda i: (i, 0)),
            pl.BlockSpec((Be, 32), lambda i: (i, 0)),
        ],
        out_shape=[
            jax.ShapeDtypeStruct((e, 64), F32),
            jax.ShapeDtypeStruct((e, 32), F32),
        ],
    )(edge_attr, ew, Ws1, b1, Ws2, b2, Ws3p, b3p)


_NC = 2
_NS = 16
_K = 40


def _make_sc_pass(n_pad, e, W, Wea, slotmap):
    nslots = W // 16
    ew_per = e // (_NC * _NS)
    rows = n_pad // _NS
    mesh = plsc.VectorSubcoreMesh(core_axis_name="c", subcore_axis_name="s")

    def body(f_hbm, ea_hbm, dst_hbm, src_hbm, zero_hbm, out_hbm,
             dst_v, src_v, ea_v, g_v, o_v, acc):
        cid = lax.axis_index("c")
        sid = lax.axis_index("s")
        pltpu.sync_copy(zero_hbm.at[pl.ds(sid * rows, rows)],
                        acc.at[pl.ds(sid * rows, rows)])
        plsc.subcore_barrier()
        wid = sid * _NC + cid
        base = wid * ew_per

        @pl.loop(0, ew_per // _K)
        def _(j):
            eb = base + j * _K
            pltpu.sync_copy(dst_hbm.at[pl.ds(eb, _K)], dst_v)
            pltpu.sync_copy(src_hbm.at[pl.ds(eb, _K)], src_v)
            pltpu.sync_copy(ea_hbm.at[pl.ds(eb, _K)], ea_v)
            pltpu.sync_copy(f_hbm.at[dst_v], g_v)
            for k in range(_K):
                regs = [ea_v[k, pl.ds(16 * r, 16)] for r in range(Wea // 16)]
                for s in range(nslots):
                    o_v[k, pl.ds(16 * s, 16)] = (
                        g_v[k, pl.ds(16 * s, 16)] * regs[slotmap[s]])
            pltpu.sync_copy(o_v, acc.at[src_v], add=True)

        plsc.subcore_barrier()
        pltpu.sync_copy(acc.at[pl.ds(sid * rows, rows)],
                        out_hbm.at[cid, pl.ds(sid * rows, rows)])

    return pl.kernel(
        body,
        out_type=jax.ShapeDtypeStruct((_NC, n_pad, W), F32),
        mesh=mesh,
        scratch_types=[
            pltpu.VMEM((_K,), jnp.int32),
            pltpu.VMEM((_K,), jnp.int32),
            pltpu.VMEM((_K, Wea), F32),
            pltpu.VMEM((_K, W), F32),
            pltpu.VMEM((_K, W), F32),
            pltpu.VMEM_SHARED((n_pad, W), F32),
        ],
        compiler_params=pltpu.CompilerParams(use_tc_tiling_on_sc=False),
    )


def _c_body(f1_ref, f2_ref, a1_ref, a2_ref, xn_ref, w3_ref, w4_ref, w5_ref,
            out_ref):
    acc1 = a1_ref[0] + a1_ref[1]
    acc2 = a2_ref[0] + a2_ref[1]
    f1 = f1_ref[...]
    f2 = f2_ref[...]
    b = lambda arr, k: arr[:, 32 * k:32 * (k + 1)]

    def planes(i, a01, a02, a12, s00, s01, s02, s11, s12):
        return [i + s00, a01 + s01, a02 + s02,
                s01 - a01, i + s11, a12 + s12,
                s02 - a02, s12 - a12, i - s00 - s11]

    M = planes(*(b(acc1, k) for k in range(4)),
               *(b(acc2, k) for k in range(5)))
    Y = planes(*(b(f1, k) for k in range(4)),
               *(b(f2, k) for k in range(5)))
    Z = []
    for i in range(3):
        for l in range(3):
            z = M[3 * i] * Y[l] + Y[3 * i] * M[l]
            for j in range(1, 3):
                z = z + M[3 * i + j] * Y[3 * j + l] + Y[3 * i + j] * M[3 * j + l]
            Z.append(z)
    nz = Z[0] * Z[0]
    for p in range(1, 9):
        nz = nz + Z[p] * Z[p]
    r = 1.0 / (jnp.maximum(nz, 0.01) + 1.0)
    third = jnp.float32(1.0 / 3.0)
    zi = (Z[0] + Z[4] + Z[8]) * third
    za01 = 0.5 * (Z[1] - Z[3])
    za02 = 0.5 * (Z[2] - Z[6])
    za12 = 0.5 * (Z[5] - Z[7])
    zs00 = Z[0] - zi
    zs01 = 0.5 * (Z[1] + Z[3])
    zs02 = 0.5 * (Z[2] + Z[6])
    zs11 = Z[4] - zi
    zs12 = 0.5 * (Z[5] + Z[7])
    w3 = w3_ref[...]
    w4 = w4_ref[...]
    w5 = w5_ref[...]
    mm = lambda x, w: jnp.dot(x * r, w, preferred_element_type=F32)
    di = mm(zi, w3)
    da01 = mm(za01, w4)
    da02 = mm(za02, w4)
    da12 = mm(za12, w4)
    ds00 = mm(zs00, w5)
    ds01 = mm(zs01, w5)
    ds02 = mm(zs02, w5)
    ds11 = mm(zs11, w5)
    ds12 = mm(zs12, w5)
    dX = [di + ds00, da01 + ds01, da02 + ds02,
          ds01 - da01, di + ds11, da12 + ds12,
          ds02 - da02, ds12 - da12, di - ds00 - ds11]
    for p in range(9):
        out_ref[p] = xn_ref[p] + dX[p]


def _run_c(F1, F2, acc1, acc2, Xn, Wt3, Wt4, Wt5, Bn):
    n = F1.shape[0]
    grid = n // Bn
    full = lambda *s: pl.BlockSpec(s, lambda i: tuple(0 for _ in s))
    return pl.pallas_call(
        _c_body,
        grid=(grid,),
        in_specs=[
            pl.BlockSpec((Bn, 128), lambda i: (i, 0)),
            pl.BlockSpec((Bn, 160), lambda i: (i, 0)),
            pl.BlockSpec((_NC, Bn, 128), lambda i: (0, i, 0)),
            pl.BlockSpec((_NC, Bn, 160), lambda i: (0, i, 0)),
            pl.BlockSpec((9, Bn, 32), lambda i: (0, i, 0)),
            full(32, 32), full(32, 32), full(32, 32),
        ],
        out_specs=pl.BlockSpec((9, Bn, 32), lambda i: (0, i, 0)),
        out_shape=jax.ShapeDtypeStruct((9, n, 32), F32),
    )(F1, F2, acc1, acc2, Xn, Wt3, Wt4, Wt5)


_SLOT1 = (0, 1, 2, 3, 2, 3, 2, 3)
_SLOT2 = (0, 1) * 5
_PERM = np.arange(96).reshape(32, 3).T.reshape(-1)


def kernel(X, edge_index, edge_weight, edge_attr, W_proj, b_proj,
           Ws1, b1, Ws2, b2, Ws3, b3, Wt0, Wt1, Wt2, Wt3, Wt4, Wt5):
    n = X.shape[0]
    hid = X.shape[1]
    e = edge_index.shape[1]

    Xt = X.reshape(n, hid, 9).transpose(2, 0, 1)
    F1, F2, Xn = _run_a(Xt, W_proj, b_proj.reshape(1, -1),
                        Wt0, Wt1, Wt2, Bn=1000)

    EA1, EA2 = _run_b(edge_attr, edge_weight.reshape(-1, 1),
                      Ws1, b1.reshape(1, -1), Ws2, b2.reshape(1, -1),
                      Ws3[:, _PERM], b3[_PERM].reshape(1, -1), Be=2000)

    src = edge_index[0]
    dst = edge_index[1]
    n_pad = ((n + 127) // 128) * 128
    z1 = jnp.zeros((n_pad, 128), F32)
    z2 = jnp.zeros((n_pad, 160), F32)
    acc1 = _make_sc_pass(n_pad, e, 128, 64, _SLOT1)(F1, EA1, dst, src, z1)
    acc2 = _make_sc_pass(n_pad, e, 160, 32, _SLOT2)(F2, EA2, dst, src, z2)

    out = _run_c(F1, F2, acc1, acc2, Xn, Wt3, Wt4, Wt5, Bn=1000)
    return out.transpose(1, 2, 0).reshape(n, W_proj.shape[1], 3, 3)


# speedup vs baseline: 55.2742x; 1.2851x over previous
import functools

import jax
import jax.numpy as jnp
import numpy as np
from jax import lax
from jax.experimental import pallas as pl
from jax.experimental.pallas import tpu as pltpu
from jax.experimental.pallas import tpu_sc as plsc

F32 = jnp.float32
_CUT = 5.0


def _a_body(xt_ref, wp_ref, bp_ref, w0_ref, w1_ref, w2_ref,
            f1_ref, f2_ref, f3_ref, xn_ref):
    wp = wp_ref[...]
    bp = bp_ref[...]
    xp = [jnp.dot(xt_ref[p], wp, preferred_element_type=F32) + bp
          for p in range(9)]
    nrm = xp[0] * xp[0]
    for p in range(1, 9):
        nrm = nrm + xp[p] * xp[p]
    inv = 1.0 / (jnp.maximum(nrm, 0.01) + 1.0)
    xn = [xp[p] * inv for p in range(9)]
    for p in range(9):
        xn_ref[p] = xn[p]
    third = jnp.float32(1.0 / 3.0)
    iso = (xn[0] + xn[4] + xn[8]) * third
    a01 = 0.5 * (xn[1] - xn[3])
    a02 = 0.5 * (xn[2] - xn[6])
    a12 = 0.5 * (xn[5] - xn[7])
    s00 = xn[0] - iso
    s01 = 0.5 * (xn[1] + xn[3])
    s02 = 0.5 * (xn[2] + xn[6])
    s11 = xn[4] - iso
    s12 = 0.5 * (xn[5] + xn[7])
    w0 = w0_ref[...]
    w1 = w1_ref[...]
    w2 = w2_ref[...]
    mm = lambda x, w: jnp.dot(x, w, preferred_element_type=F32)
    f1_ref[...] = jnp.concatenate(
        [mm(iso, w0), mm(a01, w1), mm(a02, w1)], axis=1)
    f2_ref[...] = jnp.concatenate(
        [mm(a12, w1), mm(s00, w2), mm(s01, w2)], axis=1)
    f3_ref[...] = jnp.concatenate(
        [mm(s02, w2), mm(s11, w2), mm(s12, w2)], axis=1)


def _run_a(Xt, W_proj, b_proj, Wt0, Wt1, Wt2, Bn):
    n = Xt.shape[1]
    grid = n // Bn
    full = lambda *s: pl.BlockSpec(s, lambda i: tuple(0 for _ in s))
    return pl.pallas_call(
        _a_body,
        grid=(grid,),
        in_specs=[
            pl.BlockSpec((9, Bn, 128), lambda i: (0, i, 0)),
            full(128, 32), full(1, 32), full(32, 32), full(32, 32),
            full(32, 32),
        ],
        out_specs=[
            pl.BlockSpec((Bn, 96), lambda i: (i, 0)),
            pl.BlockSpec((Bn, 96), lambda i: (i, 0)),
            pl.BlockSpec((Bn, 96), lambda i: (i, 0)),
            pl.BlockSpec((9, Bn, 32), lambda i: (0, i, 0)),
        ],
        out_shape=[
            jax.ShapeDtypeStruct((n, 96), F32),
            jax.ShapeDtypeStruct((n, 96), F32),
            jax.ShapeDtypeStruct((n, 96), F32),
            jax.ShapeDtypeStruct((9, n, 32), F32),
        ],
    )(Xt, W_proj, b_proj, Wt0, Wt1, Wt2)


def _b_body(attr_ref, ew_ref, w1_ref, b1_ref, w2_ref, b2_ref, w3_ref, b3_ref,
            ea1_ref, ea2_ref, ea3_ref):
    mm = lambda x, w: jnp.dot(x, w, preferred_element_type=F32)
    silu = lambda x: x * jax.nn.sigmoid(x)
    h = silu(mm(attr_ref[...], w1_ref[...]) + b1_ref[...])
    h = silu(mm(h, w2_ref[...]) + b2_ref[...])
    h = silu(mm(h, w3_ref[...]) + b3_ref[...])
    d = ew_ref[...]
    c = jnp.where(d < _CUT, 0.5 * (jnp.cos(d * (np.pi / _CUT)) + 1.0), 0.0)
    h = h * c
    ea1_ref[...] = h[:, :64]
    ea2_ref[...] = h[:, 32:96]
    ea3_ref[...] = h[:, 64:96]


def _run_b(edge_attr, ew, Ws1, b1, Ws2, b2, Ws3p, b3p, Be):
    e = edge_attr.shape[0]
    grid = e // Be
    full = lambda *s: pl.BlockSpec(s, lambda i: tuple(0 for _ in s))
    return pl.pallas_call(
        _b_body,
        grid=(grid,),
        in_specs=[
            pl.BlockSpec((Be, 32), lambda i: (i, 0)),
            pl.BlockSpec((Be, 1), lambda i: (i, 0)),
            full(32, 32), full(1, 32), full(32, 64), full(1, 64),
            full(64, 96), full(1, 96),
        ],
        out_specs=[
            pl.BlockSpec((Be, 64), lambda i: (i, 0)),
            pl.BlockSpec((Be, 64), lambda i: (i, 0)),
            pl.BlockSpec((Be, 32), lambda i: (i, 0)),
        ],
        out_shape=[
            jax.ShapeDtypeStruct((e, 64), F32),
            jax.ShapeDtypeStruct((e, 64), F32),
            jax.ShapeDtypeStruct((e, 32), F32),
        ],
    )(edge_attr, ew, Ws1, b1, Ws2, b2, Ws3p, b3p)


_NC = 2
_NS = 16
_K = 200
_KSUB = 40


def _make_sc_pass(n_pad, e, W, Wea, slotmap):
    nslots = W // 16
    ew_per = e // (_NC * _NS)
    rows = n_pad // _NS
    mesh = plsc.VectorSubcoreMesh(core_axis_name="c", subcore_axis_name="s")

    def body(f_hbm, ea_hbm, dst_hbm, src_hbm, zero_hbm, out_hbm,
             dst_v, src_v, ea_v, g_v, acc, sem_d, sem_s, sem_e):
        cid = lax.axis_index("c")
        sid = lax.axis_index("s")
        pltpu.sync_copy(zero_hbm.at[pl.ds(sid * rows, rows)],
                        acc.at[pl.ds(sid * rows, rows)])
        plsc.subcore_barrier()
        wid = sid * _NC + cid
        base = wid * ew_per

        @pl.loop(0, ew_per // _K)
        def _(j):
            eb = base + j * _K
            cp_d = pltpu.async_copy(dst_hbm.at[pl.ds(eb, _K)], dst_v, sem_d)
            cp_s = pltpu.async_copy(src_hbm.at[pl.ds(eb, _K)], src_v, sem_s)
            cp_e = pltpu.async_copy(ea_hbm.at[pl.ds(eb, _K)], ea_v, sem_e)
            cp_d.wait()
            pltpu.sync_copy(f_hbm.at[dst_v], g_v)
            cp_e.wait()

            @pl.loop(0, _K, step=_KSUB)
            def _(k0):
                for kk in range(_KSUB):
                    k = k0 + kk
                    regs = [ea_v[k, pl.ds(16 * r, 16)]
                            for r in range(Wea // 16)]
                    for s in range(nslots):
                        g_v[k, pl.ds(16 * s, 16)] = (
                            g_v[k, pl.ds(16 * s, 16)] * regs[slotmap[s]])

            cp_s.wait()
            pltpu.sync_copy(g_v, acc.at[src_v], add=True)

        plsc.subcore_barrier()
        pltpu.sync_copy(acc.at[pl.ds(sid * rows, rows)],
                        out_hbm.at[cid, pl.ds(sid * rows, rows)])

    return pl.kernel(
        body,
        out_type=jax.ShapeDtypeStruct((_NC, n_pad, W), F32),
        mesh=mesh,
        scratch_types=[
            pltpu.VMEM((_K,), jnp.int32),
            pltpu.VMEM((_K,), jnp.int32),
            pltpu.VMEM((_K, Wea), F32),
            pltpu.VMEM((_K, W), F32),
            pltpu.VMEM_SHARED((n_pad, W), F32),
            pltpu.SemaphoreType.DMA,
            pltpu.SemaphoreType.DMA,
            pltpu.SemaphoreType.DMA,
        ],
        compiler_params=pltpu.CompilerParams(use_tc_tiling_on_sc=False),
    )


def _c_body(f1_ref, f2_ref, f3_ref, a1_ref, a2_ref, a3_ref, xn_ref,
            w3_ref, w4_ref, w5_ref, out_ref):
    acc1 = a1_ref[0] + a1_ref[1]
    acc2 = a2_ref[0] + a2_ref[1]
    acc3 = a3_ref[0] + a3_ref[1]
    f1 = f1_ref[...]
    f2 = f2_ref[...]
    f3 = f3_ref[...]
    b = lambda arr, k: arr[:, 32 * k:32 * (k + 1)]

    def planes(i, a01, a02, a12, s00, s01, s02, s11, s12):
        return [i + s00, a01 + s01, a02 + s02,
                s01 - a01, i + s11, a12 + s12,
                s02 - a02, s12 - a12, i - s00 - s11]

    M = planes(*(b(acc1, k) for k in range(3)),
               *(b(acc2, k) for k in range(3)),
               *(b(acc3, k) for k in range(3)))
    Y = planes(*(b(f1, k) for k in range(3)),
               *(b(f2, k) for k in range(3)),
               *(b(f3, k) for k in range(3)))
    Z = []
    for i in range(3):
        for l in range(3):
            z = M[3 * i] * Y[l] + Y[3 * i] * M[l]
            for j in range(1, 3):
                z = z + M[3 * i + j] * Y[3 * j + l] + Y[3 * i + j] * M[3 * j + l]
            Z.append(z)
    nz = Z[0] * Z[0]
    for p in range(1, 9):
        nz = nz + Z[p] * Z[p]
    r = 1.0 / (jnp.maximum(nz, 0.01) + 1.0)
    third = jnp.float32(1.0 / 3.0)
    zi = (Z[0] + Z[4] + Z[8]) * third
    za01 = 0.5 * (Z[1] - Z[3])
    za02 = 0.5 * (Z[2] - Z[6])
    za12 = 0.5 * (Z[5] - Z[7])
    zs00 = Z[0] - zi
    zs01 = 0.5 * (Z[1] + Z[3])
    zs02 = 0.5 * (Z[2] + Z[6])
    zs11 = Z[4] - zi
    zs12 = 0.5 * (Z[5] + Z[7])
    w3 = w3_ref[...]
    w4 = w4_ref[...]
    w5 = w5_ref[...]
    mm = lambda x, w: jnp.dot(x * r, w, preferred_element_type=F32)
    di = mm(zi, w3)
    da01 = mm(za01, w4)
    da02 = mm(za02, w4)
    da12 = mm(za12, w4)
    ds00 = mm(zs00, w5)
    ds01 = mm(zs01, w5)
    ds02 = mm(zs02, w5)
    ds11 = mm(zs11, w5)
    ds12 = mm(zs12, w5)
    dX = [di + ds00, da01 + ds01, da02 + ds02,
          ds01 - da01, di + ds11, da12 + ds12,
          ds02 - da02, ds12 - da12, di - ds00 - ds11]
    for p in range(9):
        out_ref[p] = xn_ref[p] + dX[p]


def _run_c(F1, F2, F3, acc1, acc2, acc3, Xn, Wt3, Wt4, Wt5, Bn):
    n = F1.shape[0]
    grid = n // Bn
    full = lambda *s: pl.BlockSpec(s, lambda i: tuple(0 for _ in s))
    return pl.pallas_call(
        _c_body,
        grid=(grid,),
        in_specs=[
            pl.BlockSpec((Bn, 96), lambda i: (i, 0)),
            pl.BlockSpec((Bn, 96), lambda i: (i, 0)),
            pl.BlockSpec((Bn, 96), lambda i: (i, 0)),
            pl.BlockSpec((_NC, Bn, 96), lambda i: (0, i, 0)),
            pl.BlockSpec((_NC, Bn, 96), lambda i: (0, i, 0)),
            pl.BlockSpec((_NC, Bn, 96), lambda i: (0, i, 0)),
            pl.BlockSpec((9, Bn, 32), lambda i: (0, i, 0)),
            full(32, 32), full(32, 32), full(32, 32),
        ],
        out_specs=pl.BlockSpec((9, Bn, 32), lambda i: (0, i, 0)),
        out_shape=jax.ShapeDtypeStruct((9, n, 32), F32),
    )(F1, F2, F3, acc1, acc2, acc3, Xn, Wt3, Wt4, Wt5)


_SLOT1 = (0, 1, 2, 3, 2, 3)
_SLOT2 = (0, 1, 2, 3, 2, 3)
_SLOT3 = (0, 1) * 3
_PERM = np.arange(96).reshape(32, 3).T.reshape(-1)


def kernel(X, edge_index, edge_weight, edge_attr, W_proj, b_proj,
           Ws1, b1, Ws2, b2, Ws3, b3, Wt0, Wt1, Wt2, Wt3, Wt4, Wt5):
    n = X.shape[0]
    hid = X.shape[1]
    e = edge_index.shape[1]

    Xt = X.reshape(n, hid, 9).transpose(2, 0, 1)
    F1, F2, F3, Xn = _run_a(Xt, W_proj, b_proj.reshape(1, -1),
                            Wt0, Wt1, Wt2, Bn=1000)

    EA1, EA2, EA3 = _run_b(edge_attr, edge_weight.reshape(-1, 1),
                           Ws1, b1.reshape(1, -1), Ws2, b2.reshape(1, -1),
                           Ws3[:, _PERM], b3[_PERM].reshape(1, -1), Be=2000)

    src = edge_index[0]
    dst = edge_index[1]
    n_pad = ((n + 127) // 128) * 128
    z = jnp.zeros((n_pad, 96), F32)
    acc1 = _make_sc_pass(n_pad, e, 96, 64, _SLOT1)(F1, EA1, dst, src, z)
    acc2 = _make_sc_pass(n_pad, e, 96, 64, _SLOT2)(F2, EA2, dst, src, z)
    acc3 = _make_sc_pass(n_pad, e, 96, 32, _SLOT3)(F3, EA3, dst, src, z)

    out = _run_c(F1, F2, F3, acc1, acc2, acc3, Xn, Wt3, Wt4, Wt5, Bn=1000)
    return out.transpose(1, 2, 0).reshape(n, W_proj.shape[1], 3, 3)
